# baseline (speedup 1.0000x reference)
import functools

import jax
import jax.numpy as jnp
from jax.experimental import pallas as pl
from jax.experimental.pallas import tpu as pltpu

B, T, DIM = 1, 2048, 1024
HDIM = 4 * DIM
E = 8
TOPK = 2
N = B * T
TILE = 128
NT = N // TILE


def _router_kernel(x_ref, rw_ref, gates_ref, lb_ref, z_ref):
    x = x_ref[...]
    logits = jax.lax.dot_general(
        x, rw_ref[...], (((1,), (1,)), ((), ())),
        preferred_element_type=jnp.float32)
    m = jnp.max(logits, axis=-1, keepdims=True)
    ex = jnp.exp(logits - m)
    se = jnp.sum(ex, axis=-1, keepdims=True)
    probs = ex / se

    lane = jax.lax.broadcasted_iota(jnp.int32, (N, E), 1)
    m1 = jnp.max(probs, axis=-1, keepdims=True)
    i1 = jnp.min(jnp.where(probs == m1, lane, E), axis=-1, keepdims=True)
    oh1 = (lane == i1)
    pm = jnp.where(oh1, -jnp.inf, probs)
    m2 = jnp.max(pm, axis=-1, keepdims=True)
    i2 = jnp.min(jnp.where(pm == m2, lane, E), axis=-1, keepdims=True)
    oh2 = (lane == i2)
    denom = m1 + m2
    gates = (jnp.where(oh1, m1, 0.0) + jnp.where(oh2, m2, 0.0)) / denom
    gates_ref[...] = gates

    counts = jnp.sum((oh1 | oh2).astype(jnp.float32), axis=0)
    f = counts / (N * TOPK)
    p_mean = jnp.mean(probs, axis=0)
    lb_ref[...] = (E * jnp.sum(f * p_mean))[None, None]

    lse = jnp.log(se[:, 0]) + m[:, 0]
    z_ref[...] = jnp.mean(jnp.square(lse))[None, None]


def _expert_kernel(x_ref, wup_ref, wdn_ref, gates_ref, out_ref, acc_ref):
    e = pl.program_id(0)
    t = pl.program_id(1)
    rows = pl.ds(t * TILE, TILE)

    xt = x_ref[rows, :]
    h = jnp.dot(xt, wup_ref[0], preferred_element_type=jnp.float32)
    h = jnp.square(jnp.maximum(h, 0.0))
    y = jnp.dot(h.astype(jnp.bfloat16), wdn_ref[0],
                preferred_element_type=jnp.float32)
    gt = gates_ref[rows, :]
    lane = jax.lax.broadcasted_iota(jnp.int32, (TILE, E + 1), 1)
    g = jnp.sum(jnp.where(lane == e, gt, 0.0), axis=1, keepdims=True)
    y = y * g

    @pl.when(e == 0)
    def _():
        acc_ref[rows, :] = y

    @pl.when(e > 0)
    def _():
        acc_ref[rows, :] += y

    @pl.when(e == E)
    def _():
        out_ref[...] = acc_ref[rows, :]


def kernel(x, router_w, W_shared_up, W_shared_down, W_up, W_down):
    xf = x.reshape(N, DIM)

    gates, lb, z = pl.pallas_call(
        _router_kernel,
        out_shape=[
            jax.ShapeDtypeStruct((N, E), jnp.float32),
            jax.ShapeDtypeStruct((1, 1), jnp.float32),
            jax.ShapeDtypeStruct((1, 1), jnp.float32),
        ],
    )(xf, router_w)

    wup_all = jnp.concatenate(
        [W_up, W_shared_up[None]], axis=0).astype(jnp.bfloat16)
    wdn_t_all = jnp.concatenate(
        [W_down, W_shared_down[None]], axis=0
    ).transpose(0, 2, 1).astype(jnp.bfloat16)
    ones = jnp.ones((N, 1), jnp.float32)
    gates_all = jnp.concatenate([gates, ones], axis=1)
    x_bf = xf.astype(jnp.bfloat16)

    out = pl.pallas_call(
        _expert_kernel,
        grid=(E + 1, NT),
        in_specs=[
            pl.BlockSpec((N, DIM), lambda e, t: (0, 0)),
            pl.BlockSpec((1, DIM, HDIM), lambda e, t: (e, 0, 0)),
            pl.BlockSpec((1, HDIM, DIM), lambda e, t: (e, 0, 0)),
            pl.BlockSpec((N, E + 1), lambda e, t: (0, 0)),
        ],
        out_specs=pl.BlockSpec((TILE, DIM), lambda e, t: (t, 0)),
        out_shape=jax.ShapeDtypeStruct((N, DIM), jnp.float32),
        scratch_shapes=[pltpu.VMEM((N, DIM), jnp.float32)],
    )(x_bf, wup_all, wdn_t_all, gates_all)

    return out.reshape(B, T, DIM), lb[0, 0], z[0, 0]


# speedup vs baseline: 1.0386x; 1.0386x over previous
import functools

import jax
import jax.numpy as jnp
from jax.experimental import pallas as pl
from jax.experimental.pallas import tpu as pltpu

B, T, DIM = 1, 2048, 1024
HDIM = 4 * DIM
E = 8
TOPK = 2
N = B * T
TILE = 256
NC = 2
NT = N // TILE // NC
NH = N // NC


def _router_kernel(x_ref, rw_ref, gates_ref, lb_ref, z_ref):
    x = x_ref[...]
    logits = jax.lax.dot_general(
        x, rw_ref[...], (((1,), (1,)), ((), ())),
        preferred_element_type=jnp.float32)
    m = jnp.max(logits, axis=-1, keepdims=True)
    ex = jnp.exp(logits - m)
    se = jnp.sum(ex, axis=-1, keepdims=True)
    probs = ex / se

    lane = jax.lax.broadcasted_iota(jnp.int32, (N, E), 1)
    m1 = jnp.max(probs, axis=-1, keepdims=True)
    i1 = jnp.min(jnp.where(probs == m1, lane, E), axis=-1, keepdims=True)
    oh1 = (lane == i1)
    pm = jnp.where(oh1, -jnp.inf, probs)
    m2 = jnp.max(pm, axis=-1, keepdims=True)
    i2 = jnp.min(jnp.where(pm == m2, lane, E), axis=-1, keepdims=True)
    oh2 = (lane == i2)
    denom = m1 + m2
    gates = (jnp.where(oh1, m1, 0.0) + jnp.where(oh2, m2, 0.0)) / denom
    gates_ref[...] = gates

    counts = jnp.sum((oh1 | oh2).astype(jnp.float32), axis=0)
    f = counts / (N * TOPK)
    p_mean = jnp.mean(probs, axis=0)
    lb_ref[...] = (E * jnp.sum(f * p_mean))[None, None]

    lse = jnp.log(se[:, 0]) + m[:, 0]
    z_ref[...] = jnp.mean(jnp.square(lse))[None, None]


def _expert_kernel(x_ref, wup_ref, wdn_ref, gates_ref, out_ref, acc_ref):
    e = pl.program_id(1)
    t = pl.program_id(2)
    rows = pl.ds(t * TILE, TILE)

    xt = x_ref[rows, :]
    h = jnp.dot(xt, wup_ref[0], preferred_element_type=jnp.float32)
    h = jnp.square(jnp.maximum(h, 0.0))
    y = jnp.dot(h.astype(jnp.bfloat16), wdn_ref[0],
                preferred_element_type=jnp.float32)
    gt = gates_ref[rows, :]
    lane = jax.lax.broadcasted_iota(jnp.int32, (TILE, E + 1), 1)
    g = jnp.sum(jnp.where(lane == e, gt, 0.0), axis=1, keepdims=True)
    y = y * g

    @pl.when(e == 0)
    def _():
        acc_ref[rows, :] = y

    @pl.when(e > 0)
    def _():
        acc_ref[rows, :] += y

    @pl.when(e == E)
    def _():
        out_ref[...] = acc_ref[rows, :]


def kernel(x, router_w, W_shared_up, W_shared_down, W_up, W_down):
    xf = x.reshape(N, DIM)

    gates, lb, z = pl.pallas_call(
        _router_kernel,
        out_shape=[
            jax.ShapeDtypeStruct((N, E), jnp.float32),
            jax.ShapeDtypeStruct((1, 1), jnp.float32),
            jax.ShapeDtypeStruct((1, 1), jnp.float32),
        ],
    )(xf, router_w)

    wup_all = jnp.concatenate(
        [W_up, W_shared_up[None]], axis=0).astype(jnp.bfloat16)
    wdn_t_all = jnp.concatenate(
        [W_down, W_shared_down[None]], axis=0
    ).transpose(0, 2, 1).astype(jnp.bfloat16)
    ones = jnp.ones((N, 1), jnp.float32)
    gates_all = jnp.concatenate([gates, ones], axis=1)
    x_bf = xf.astype(jnp.bfloat16)

    out = pl.pallas_call(
        _expert_kernel,
        grid=(NC, E + 1, NT),
        in_specs=[
            pl.BlockSpec((NH, DIM), lambda c, e, t: (c, 0)),
            pl.BlockSpec((1, DIM, HDIM), lambda c, e, t: (e, 0, 0)),
            pl.BlockSpec((1, HDIM, DIM), lambda c, e, t: (e, 0, 0)),
            pl.BlockSpec((NH, E + 1), lambda c, e, t: (c, 0)),
        ],
        out_specs=pl.BlockSpec((TILE, DIM), lambda c, e, t: (c * NT + t, 0)),
        out_shape=jax.ShapeDtypeStruct((N, DIM), jnp.float32),
        scratch_shapes=[pltpu.VMEM((NH, DIM), jnp.float32)],
        compiler_params=pltpu.CompilerParams(
            dimension_semantics=("parallel", "arbitrary", "arbitrary")),
    )(x_bf, wup_all, wdn_t_all, gates_all)

    return out.reshape(B, T, DIM), lb[0, 0], z[0, 0]
